# Initial kernel scaffold; baseline (speedup 1.0000x reference)
import jax
import jax.numpy as jnp
from jax.experimental import pallas as pl


def kernel(x, W_v, W_qk):
    raise NotImplementedError("write your pallas kernel here")


# speedup vs baseline: 6.1296x; 6.1296x over previous
import jax
import jax.numpy as jnp
from jax import lax
from jax.experimental import pallas as pl
from jax.experimental.pallas import tpu as pltpu

_HEADS = 12
_DH = 64
_N = 4096
_D = 768
_LOGN = 12
_TILE = 512
_NT = _N // _TILE


def _mm_t(a, b):
    return lax.dot_general(a, b, (((1,), (1,)), ((), ())),
                           preferred_element_type=jnp.float32)


def _swap_pairs(x, stride):
    n, c = x.shape
    if stride >= 8:
        g = n // (2 * stride)
        x4 = x.reshape(g, 2, stride, c)
        sw = jnp.concatenate([x4[:, 1:2], x4[:, 0:1]], axis=1)
        return sw.reshape(n, c)
    down = pltpu.roll(x, stride, 0)
    up = pltpu.roll(x, n - stride, 0)
    rows = lax.broadcasted_iota(jnp.int32, (n, 1), 0)
    bit = (rows & stride) != 0
    return jnp.where(bit, down, up)


def _v_kernel(x_ref, wv_ref, o_ref):
    o_ref[...] = _mm_t(x_ref[...], wv_ref[...])


def _stage_kernel(v_ref, wqk_ref, out_ref, hs_ref, e_ref, es_ref):
    s = pl.program_id(0)

    @pl.when(s == 0)
    def _():
        out_ref[...] = v_ref[...]

    for c in range(_LOGN):
        @pl.when(s == c)
        def _(c=c):
            hs_ref[...] = _swap_pairs(out_ref[...], 1 << c)

    wq = wqk_ref[0, :_D, :]
    wk = wqk_ref[0, _D:, :]

    scale = jnp.float32(_DH ** -0.5)
    seg = (lax.broadcasted_iota(jnp.int32, (_D, _HEADS), 0) // _DH ==
           lax.broadcasted_iota(jnp.int32, (_D, _HEADS), 1))
    smat = jnp.where(seg, scale, jnp.float32(0.0))

    for t in range(_NT):
        rows = pl.ds(t * _TILE, _TILE)
        h_t = out_ref[rows, :]
        hs_t = hs_ref[rows, :]
        gidx = t * _TILE + lax.broadcasted_iota(jnp.int32, (_TILE, 1), 0)
        bit = lax.shift_right_logical(gidx, s) & 1
        ha_t = jnp.where(bit == 1, hs_t, h_t)
        qa_t = _mm_t(ha_t, wq)
        k_t = _mm_t(h_t, wk)
        e_ref[rows, :] = lax.dot_general(
            qa_t * k_t, smat, (((1,), (0,)), ((), ())),
            preferred_element_type=jnp.float32)

    for c in range(_LOGN):
        @pl.when(s == c)
        def _(c=c):
            es_ref[...] = _swap_pairs(e_ref[...], 1 << c)

    for c in range(_LOGN):
        @pl.when(s == c)
        def _(c=c):
            hs_ref[...] = _swap_pairs(v_ref[...], 1 << c)

    rep = (lax.broadcasted_iota(jnp.int32, (_HEADS, _D), 0) ==
           lax.broadcasted_iota(jnp.int32, (_HEADS, _D), 1) // _DH)
    bmat = jnp.where(rep, jnp.float32(1.0), jnp.float32(0.0))

    for t in range(_NT):
        rows = pl.ds(t * _TILE, _TILE)
        w_t = jax.nn.sigmoid(e_ref[rows, :] - es_ref[rows, :])
        wf_t = lax.dot_general(w_t, bmat, (((1,), (0,)), ((), ())),
                               preferred_element_type=jnp.float32)
        out_ref[rows, :] = wf_t * (v_ref[rows, :] + hs_ref[rows, :])


def _run(x2, W_v, W_qk, interpret=False):
    v = pl.pallas_call(
        _v_kernel,
        grid=(_NT,),
        in_specs=[pl.BlockSpec((_TILE, _D), lambda i: (i, 0)),
                  pl.BlockSpec((_D, _D), lambda i: (0, 0))],
        out_specs=pl.BlockSpec((_TILE, _D), lambda i: (i, 0)),
        out_shape=jax.ShapeDtypeStruct((_N, _D), jnp.float32),
        interpret=interpret,
    )(x2, W_v)

    h = pl.pallas_call(
        _stage_kernel,
        grid=(_LOGN,),
        in_specs=[pl.BlockSpec((_N, _D), lambda s: (0, 0)),
                  pl.BlockSpec((1, 2 * _D, _D), lambda s: (s, 0, 0))],
        out_specs=pl.BlockSpec((_N, _D), lambda s: (0, 0)),
        out_shape=jax.ShapeDtypeStruct((_N, _D), jnp.float32),
        scratch_shapes=[pltpu.VMEM((_N, _D), jnp.float32),
                        pltpu.VMEM((_N, _HEADS), jnp.float32),
                        pltpu.VMEM((_N, _HEADS), jnp.float32)],
        interpret=interpret,
    )(v, W_qk)
    return h


def kernel(x, W_v, W_qk):
    B, N, D = x.shape
    h = _run(x.reshape(N, D), W_v, W_qk)
    return h.reshape(B, N, D)


# speedup vs baseline: 6.6371x; 1.0828x over previous
import jax
import jax.numpy as jnp
from jax import lax
from jax.experimental import pallas as pl
from jax.experimental.pallas import tpu as pltpu

_HEADS = 12
_DH = 64
_N = 4096
_D = 768
_LOGN = 12
_TILE = 512
_NT = _N // _TILE


def _mm_t(a, b):
    return lax.dot_general(a, b, (((1,), (1,)), ((), ())),
                           preferred_element_type=jnp.float32)


def _mm(a, b):
    return lax.dot_general(a, b, (((1,), (0,)), ((), ())),
                           preferred_element_type=jnp.float32)


def _swap_pairs(x, stride):
    n, c = x.shape
    if stride >= 8:
        g = n // (2 * stride)
        x4 = x.reshape(g, 2, stride, c)
        sw = jnp.concatenate([x4[:, 1:2], x4[:, 0:1]], axis=1)
        return sw.reshape(n, c)
    down = pltpu.roll(x, stride, 0)
    up = pltpu.roll(x, n - stride, 0)
    rows = lax.broadcasted_iota(jnp.int32, (n, 1), 0)
    bit = (rows & stride) != 0
    return jnp.where(bit, down, up)


def _v_kernel(x_ref, wv_ref, o_ref):
    o_ref[...] = _mm_t(x_ref[...].astype(jnp.bfloat16), wv_ref[...])


def _stage_kernel(v_ref, wqk_ref, out_ref, hb_ref, hs_ref, vb_ref,
                  e_ref, es_ref):
    s = pl.program_id(0)

    @pl.when(s == 0)
    def _():
        vb = v_ref[...].astype(jnp.bfloat16)
        hb_ref[...] = vb
        vb_ref[...] = vb

    for c in range(_LOGN):
        @pl.when(s == c)
        def _(c=c):
            hs_ref[...] = _swap_pairs(hb_ref[...], 1 << c)

    wq = wqk_ref[0, :_D, :]
    wk = wqk_ref[0, _D:, :]

    scale = jnp.float32(_DH ** -0.5)
    seg = (lax.broadcasted_iota(jnp.int32, (_D, _HEADS), 0) // _DH ==
           lax.broadcasted_iota(jnp.int32, (_D, _HEADS), 1))
    smat = jnp.where(seg, scale, jnp.float32(0.0)).astype(jnp.bfloat16)

    for t in range(_NT):
        rows = pl.ds(t * _TILE, _TILE)
        h_t = hb_ref[rows, :]
        hs_t = hs_ref[rows, :]
        gidx = t * _TILE + lax.broadcasted_iota(jnp.int32, (_TILE, 1), 0)
        bit = lax.shift_right_logical(gidx, s) & 1
        ha_t = jnp.where(bit == 1, hs_t, h_t)
        qa_t = _mm_t(ha_t, wq)
        k_t = _mm_t(h_t, wk)
        p_t = (qa_t * k_t).astype(jnp.bfloat16)
        e_ref[rows, :] = _mm(p_t, smat)

    for c in range(_LOGN):
        @pl.when(s == c)
        def _(c=c):
            es_ref[...] = _swap_pairs(e_ref[...], 1 << c)

    for c in range(_LOGN):
        @pl.when(s == c)
        def _(c=c):
            hs_ref[...] = _swap_pairs(vb_ref[...], 1 << c)

    rep = (lax.broadcasted_iota(jnp.int32, (_HEADS, _D), 0) ==
           lax.broadcasted_iota(jnp.int32, (_HEADS, _D), 1) // _DH)
    bmat = jnp.where(rep, jnp.float32(1.0), jnp.float32(0.0)).astype(jnp.bfloat16)

    for t in range(_NT):
        rows = pl.ds(t * _TILE, _TILE)
        w_t = jax.nn.sigmoid(e_ref[rows, :] - es_ref[rows, :])
        wf_t = _mm(w_t.astype(jnp.bfloat16), bmat)
        res = wf_t * (v_ref[rows, :] + hs_ref[rows, :].astype(jnp.float32))
        hb_ref[rows, :] = res.astype(jnp.bfloat16)
        out_ref[rows, :] = res


def _run(x2, W_v, W_qk, interpret=False):
    v = pl.pallas_call(
        _v_kernel,
        grid=(_NT,),
        in_specs=[pl.BlockSpec((_TILE, _D), lambda i: (i, 0)),
                  pl.BlockSpec((_D, _D), lambda i: (0, 0))],
        out_specs=pl.BlockSpec((_TILE, _D), lambda i: (i, 0)),
        out_shape=jax.ShapeDtypeStruct((_N, _D), jnp.float32),
        interpret=interpret,
    )(x2, W_v.astype(jnp.bfloat16))

    h = pl.pallas_call(
        _stage_kernel,
        grid=(_LOGN,),
        in_specs=[pl.BlockSpec((_N, _D), lambda s: (0, 0)),
                  pl.BlockSpec((1, 2 * _D, _D), lambda s: (s, 0, 0))],
        out_specs=pl.BlockSpec((_N, _D), lambda s: (0, 0)),
        out_shape=jax.ShapeDtypeStruct((_N, _D), jnp.float32),
        scratch_shapes=[pltpu.VMEM((_N, _D), jnp.bfloat16),
                        pltpu.VMEM((_N, _D), jnp.bfloat16),
                        pltpu.VMEM((_N, _D), jnp.bfloat16),
                        pltpu.VMEM((_N, _HEADS), jnp.float32),
                        pltpu.VMEM((_N, _HEADS), jnp.float32)],
        interpret=interpret,
    )(v, W_qk.astype(jnp.bfloat16))
    return h


def kernel(x, W_v, W_qk):
    B, N, D = x.shape
    h = _run(x.reshape(N, D), W_v, W_qk)
    return h.reshape(B, N, D)
